# baseline (speedup 1.0000x reference)
import functools

import jax
import jax.numpy as jnp
from jax import lax
from jax.experimental import pallas as pl
from jax.experimental.pallas import tpu as pltpu
from jax.experimental.pallas import tpu_sc as plsc

N = 10000
E = 320000
NC = 2
NS = 16
NW = NC * NS
CH = 128
CPW = 80
E_PAD = NW * CPW * CH
RS_A = 632
RS_B = N - 15 * RS_A
N128 = 10112


def _pad_edges(row, col, ew):
    npad = E_PAD - E
    fill = (jnp.arange(npad, dtype=jnp.int32) * 37) % N
    rowp = jnp.concatenate([row, fill])
    colp = jnp.concatenate([col, fill])
    ewp = jnp.concatenate([ew, jnp.zeros((npad,), jnp.float32)])

    def chunkify(a):
        return (a.reshape(CPW, NW, CH).transpose(1, 0, 2)
                .reshape(NW * CPW, CH))

    return chunkify(rowp), chunkify(colp), chunkify(ewp)


def _sc_deg(colc, ewc, zeros_n):
    mesh = plsc.VectorSubcoreMesh(core_axis_name="c", subcore_axis_name="s")

    @functools.partial(
        pl.kernel,
        out_type=jax.ShapeDtypeStruct((NC * N128,), jnp.float32),
        mesh=mesh,
        scratch_types=[
            pltpu.VMEM((CPW, CH), jnp.int32),
            pltpu.VMEM((CPW, CH), jnp.float32),
            pltpu.VMEM_SHARED((N128,), jnp.float32),
            pltpu.SemaphoreType.DMA,
        ],
    )
    def k(col_hbm, ew_hbm, z_hbm, out_hbm, colvs, ewvs, deg_sh, sem):
        cid = lax.axis_index("c")
        sid = lax.axis_index("s")
        w = (cid * NS + sid).astype(jnp.int32)
        sbase = pl.multiple_of(w * CPW, 8)

        @pl.when(sid == 0)
        def _():
            pltpu.sync_copy(z_hbm, deg_sh)

        pltpu.sync_copy(col_hbm.at[pl.ds(sbase, CPW)], colvs)
        pltpu.sync_copy(ew_hbm.at[pl.ds(sbase, CPW)], ewvs)
        plsc.subcore_barrier()

        def body(tt, carry):
            t0 = tt * jnp.int32(8)
            descs = [pltpu.make_async_copy(ewvs.at[t0 + j],
                                           deg_sh.at[colvs.at[t0 + j]], sem)
                     for j in range(8)]
            for dsc in descs:
                dsc.start(add=True)
            for dsc in descs:
                dsc.wait()
            return carry

        lax.fori_loop(jnp.int32(0), jnp.int32(CPW // 8), body, jnp.int32(0))
        plsc.subcore_barrier()

        @pl.when((sid == 0) & (cid == 0))
        def _():
            pltpu.sync_copy(deg_sh, out_hbm.at[pl.ds(0, N128)])

        @pl.when((sid == 0) & (cid == 1))
        def _():
            pltpu.sync_copy(deg_sh, out_hbm.at[pl.ds(N128, N128)])

    return k(colc, ewc, zeros_n).reshape(NC, N128)[:, :N]


def _sc_agg(hp, rowc, colc, ewc, zeros_nd, da):
    d = 128
    mesh = plsc.VectorSubcoreMesh(core_axis_name="c", subcore_axis_name="s")

    scaled_bufs = (
        [pltpu.VMEM((CH, da), jnp.float32),
         pltpu.VMEM((CH, da), jnp.float32)]
        if da != d else [])

    @functools.partial(
        pl.kernel,
        out_type=jax.ShapeDtypeStruct((NC * N, da), jnp.float32),
        mesh=mesh,
        scratch_types=[
            pltpu.VMEM((CPW // 2, CH), jnp.int32),
            pltpu.VMEM((CPW // 2, CH), jnp.int32),
            pltpu.VMEM((CPW // 2, CH), jnp.float32),
            pltpu.VMEM((CH, d), jnp.float32),
            pltpu.VMEM((CH, d), jnp.float32),
        ] + scaled_bufs + [
            pltpu.VMEM_SHARED((N, da), jnp.float32),
            pltpu.SemaphoreType.DMA,
            pltpu.SemaphoreType.DMA,
        ],
        compiler_params=pltpu.CompilerParams(needs_layout_passes=False),
    )
    def k(hp_hbm, row_hbm, col_hbm, ew_hbm, z_hbm, out_hbm,
          rowvs, colvs, ewvs, rows0, rows1, *rest):
        if da != d:
            sc0, sc1, acc, sem_g, sem_s = rest
        else:
            acc, sem_g, sem_s = rest
            sc0, sc1 = rows0, rows1
        cid = lax.axis_index("c")
        sid = lax.axis_index("s")
        w = (cid * NS + sid).astype(jnp.int32)
        sbase = pl.multiple_of(w * CPW, 8)
        rbase = pl.multiple_of(sid * RS_A, 8)

        @pl.when(sid < NS - 1)
        def _():
            pltpu.sync_copy(z_hbm.at[pl.ds(rbase, RS_A)],
                            acc.at[pl.ds(rbase, RS_A)])

        @pl.when(sid == NS - 1)
        def _():
            pltpu.sync_copy(z_hbm.at[pl.ds(rbase, RS_B)],
                            acc.at[pl.ds(rbase, RS_B)])

        plsc.subcore_barrier()
        HALF = CPW // 2

        def gather_desc(t, dst):
            return pltpu.make_async_copy(hp_hbm.at[rowvs.at[t]], dst, sem_g)

        def scatter_desc(t, src):
            return pltpu.make_async_copy(src, acc.at[colvs.at[t]], sem_s)

        def process(t, rb, sb, other_sb):
            gather_desc(t, rb).wait()

            @pl.when(t >= 1)
            def _():
                scatter_desc(t - jnp.int32(1), other_sb).wait()

            @pl.when(t < HALF - 1)
            def _():
                gather_desc(t + jnp.int32(1),
                            rows1 if rb is rows0 else rows0).start()

            tsplat = jnp.full((16,), 0, jnp.int32) + t

            def scale(_, j):
                s = plsc.load_gather(
                    ewvs, [tsplat, jnp.full((16,), 0, jnp.int32) + j])
                for t16 in range(da // 16):
                    sl = pl.ds(t16 * 16, 16)
                    sb[j, sl] = rb[j, sl] * s
                return j + jnp.int32(1)

            lax.fori_loop(0, CH, scale, jnp.int32(0), unroll=4)

            scatter_desc(t, sb).start(add=True)

        def run_half(h, carry):
            hbase = pl.multiple_of(sbase + h * jnp.int32(HALF), 8)
            pltpu.sync_copy(row_hbm.at[pl.ds(hbase, HALF)], rowvs)
            pltpu.sync_copy(col_hbm.at[pl.ds(hbase, HALF)], colvs)
            pltpu.sync_copy(ew_hbm.at[pl.ds(hbase, HALF)], ewvs)
            gather_desc(jnp.int32(0), rows0).start()

            def body(tt, c2):
                t = tt * jnp.int32(2)
                process(t, rows0, sc0, sc1)
                process(t + jnp.int32(1), rows1, sc1, sc0)
                return c2

            lax.fori_loop(jnp.int32(0), jnp.int32(HALF // 2), body,
                          jnp.int32(0))
            scatter_desc(jnp.int32(HALF - 1), sc1).wait()
            return carry

        lax.fori_loop(jnp.int32(0), jnp.int32(2), run_half, jnp.int32(0))
        plsc.subcore_barrier()

        obase = pl.multiple_of(cid * N + sid * RS_A, 8)

        @pl.when(sid < NS - 1)
        def _():
            pltpu.sync_copy(acc.at[pl.ds(rbase, RS_A)],
                            out_hbm.at[pl.ds(obase, RS_A)])

        @pl.when(sid == NS - 1)
        def _():
            pltpu.sync_copy(acc.at[pl.ds(rbase, RS_B)],
                            out_hbm.at[pl.ds(obase, RS_B)])

    return k(hp, rowc, colc, ewc, zeros_nd).reshape(NC, N, da)


BN = 400


def _l1_body(x_ref, w_ref, dp_ref, hp_ref, dv_ref):
    deg = dp_ref[0] + dp_ref[1] + jnp.float32(1.0)
    dinv = jnp.where(deg > 0, lax.rsqrt(deg), jnp.float32(0.0))
    h = jnp.dot(x_ref[...], w_ref[...], preferred_element_type=jnp.float32)
    hp_ref[...] = h * dinv
    dv_ref[...] = dinv


def _tc_layer1(x, W1, degp3):
    return pl.pallas_call(
        _l1_body,
        grid=(N // BN,),
        in_specs=[
            pl.BlockSpec((BN, 128), lambda i: (i, i * 0)),
            pl.BlockSpec((128, 128), lambda i: (i * 0, i * 0)),
            pl.BlockSpec((2, BN, 1), lambda i: (i * 0, i, i * 0)),
        ],
        out_specs=[
            pl.BlockSpec((BN, 128), lambda i: (i, i * 0)),
            pl.BlockSpec((BN, 1), lambda i: (i, i * 0)),
        ],
        out_shape=[
            jax.ShapeDtypeStruct((N, 128), jnp.float32),
            jax.ShapeDtypeStruct((N, 1), jnp.float32),
        ],
    )(x, W1, degp3)


def _l2_body(a_ref, hp_ref, d_ref, b_ref, w_ref, o_ref):
    agg = a_ref[0] + a_ref[1] + hp_ref[...]
    z = jax.nn.relu(agg * d_ref[...] + b_ref[...])
    h2 = jnp.dot(z, w_ref[...], preferred_element_type=jnp.float32)
    o_ref[...] = h2 * d_ref[...]


def _tc_layer2(accp1, hp1, dinv_col, b1_row, W2):
    return pl.pallas_call(
        _l2_body,
        grid=(N // BN,),
        in_specs=[
            pl.BlockSpec((2, BN, 128), lambda i: (i * 0, i, i * 0)),
            pl.BlockSpec((BN, 128), lambda i: (i, i * 0)),
            pl.BlockSpec((BN, 1), lambda i: (i, i * 0)),
            pl.BlockSpec((1, 128), lambda i: (i * 0, i * 0)),
            pl.BlockSpec((128, 128), lambda i: (i * 0, i * 0)),
        ],
        out_specs=pl.BlockSpec((BN, 128), lambda i: (i, i * 0)),
        out_shape=jax.ShapeDtypeStruct((N, 128), jnp.float32),
    )(accp1, hp1, dinv_col, b1_row, W2)


def _fin_body(a_ref, hp_ref, d_ref, b_ref, o_ref):
    agg = a_ref[0] + a_ref[1] + hp_ref[...]
    o_ref[...] = agg[:, :64] * d_ref[...] + b_ref[...]


def _tc_final(accp2, hp2, dinv_col, b2_row):
    return pl.pallas_call(
        _fin_body,
        grid=(N // BN,),
        in_specs=[
            pl.BlockSpec((2, BN, 128), lambda i: (i * 0, i, i * 0)),
            pl.BlockSpec((BN, 128), lambda i: (i, i * 0)),
            pl.BlockSpec((BN, 1), lambda i: (i, i * 0)),
            pl.BlockSpec((1, 64), lambda i: (i * 0, i * 0)),
        ],
        out_specs=pl.BlockSpec((BN, 64), lambda i: (i, i * 0)),
        out_shape=jax.ShapeDtypeStruct((N, 64), jnp.float32),
    )(accp2, hp2, dinv_col, b2_row)


def kernel(x, edge_index, edge_weight, W1, b1, W2, b2):
    row = edge_index[0].astype(jnp.int32)
    col = edge_index[1].astype(jnp.int32)
    ew = edge_weight.astype(jnp.float32)
    x = x.astype(jnp.float32)

    rowc, colc, ewc = _pad_edges(row, col, ew)

    zeros_n = jnp.zeros((N128,), jnp.float32)
    zeros_n128 = jnp.zeros((N, 128), jnp.float32)
    zeros_n64 = jnp.zeros((N, 64), jnp.float32)

    W2p = jnp.concatenate(
        [W2.astype(jnp.float32), jnp.zeros((128, 64), jnp.float32)], axis=1)

    degp = _sc_deg(colc, ewc, zeros_n)
    degp3 = degp.reshape(NC, N, 1)

    hp1, dinv_col = _tc_layer1(x, W1.astype(jnp.float32), degp3)
    accp1 = _sc_agg(hp1, rowc, colc, ewc, zeros_n128, 128)
    hp2 = _tc_layer2(accp1, hp1, dinv_col,
                     b1.astype(jnp.float32).reshape(1, 128),
                     W2p)
    accp2 = _sc_agg(hp2, rowc, colc, ewc, zeros_n128, 128)
    out = _tc_final(accp2, hp2, dinv_col,
                    b2.astype(jnp.float32).reshape(1, 64))
    return out


# speedup vs baseline: 1.0016x; 1.0016x over previous
import functools

import jax
import jax.numpy as jnp
from jax import lax
from jax.experimental import pallas as pl
from jax.experimental.pallas import tpu as pltpu
from jax.experimental.pallas import tpu_sc as plsc

N = 10000
E = 320000
NC = 2
NS = 16
NW = NC * NS
CH = 128
CPW = 80
E_PAD = NW * CPW * CH
RS_A = 632
RS_B = N - 15 * RS_A
N128 = 10112


def _pad_edges(row, col, ew):
    npad = E_PAD - E
    fill = (jnp.arange(npad, dtype=jnp.int32) * 37) % N
    rowp = jnp.concatenate([row, fill])
    colp = jnp.concatenate([col, fill])
    ewp = jnp.concatenate([ew, jnp.zeros((npad,), jnp.float32)])

    def chunkify(a):
        return (a.reshape(CPW, NW, CH).transpose(1, 0, 2)
                .reshape(NW * CPW, CH))

    return chunkify(rowp), chunkify(colp), chunkify(ewp)


def _sc_deg(colc, ewc, zeros_n):
    mesh = plsc.VectorSubcoreMesh(core_axis_name="c", subcore_axis_name="s")

    @functools.partial(
        pl.kernel,
        out_type=jax.ShapeDtypeStruct((NC * N128,), jnp.float32),
        mesh=mesh,
        scratch_types=[
            pltpu.VMEM((CPW, CH), jnp.int32),
            pltpu.VMEM((CPW, CH), jnp.float32),
            pltpu.VMEM_SHARED((N128,), jnp.float32),
            pltpu.SemaphoreType.DMA,
        ],
    )
    def k(col_hbm, ew_hbm, z_hbm, out_hbm, colvs, ewvs, deg_sh, sem):
        cid = lax.axis_index("c")
        sid = lax.axis_index("s")
        w = (cid * NS + sid).astype(jnp.int32)
        sbase = pl.multiple_of(w * CPW, 8)

        @pl.when(sid == 0)
        def _():
            pltpu.sync_copy(z_hbm, deg_sh)

        pltpu.sync_copy(col_hbm.at[pl.ds(sbase, CPW)], colvs)
        pltpu.sync_copy(ew_hbm.at[pl.ds(sbase, CPW)], ewvs)
        plsc.subcore_barrier()

        def body(tt, carry):
            t0 = tt * jnp.int32(8)
            descs = [pltpu.make_async_copy(ewvs.at[t0 + j],
                                           deg_sh.at[colvs.at[t0 + j]], sem)
                     for j in range(8)]
            for dsc in descs:
                dsc.start(add=True)
            for dsc in descs:
                dsc.wait()
            return carry

        lax.fori_loop(jnp.int32(0), jnp.int32(CPW // 8), body, jnp.int32(0))
        plsc.subcore_barrier()

        @pl.when((sid == 0) & (cid == 0))
        def _():
            pltpu.sync_copy(deg_sh, out_hbm.at[pl.ds(0, N128)])

        @pl.when((sid == 0) & (cid == 1))
        def _():
            pltpu.sync_copy(deg_sh, out_hbm.at[pl.ds(N128, N128)])

    return k(colc, ewc, zeros_n).reshape(NC, N128)[:, :N]


def _sc_agg(hp, rowc, colc, ewc, zeros_nd, da):
    d = 128
    mesh = plsc.VectorSubcoreMesh(core_axis_name="c", subcore_axis_name="s")

    scaled_bufs = (
        [pltpu.VMEM((CH, da), jnp.float32),
         pltpu.VMEM((CH, da), jnp.float32)]
        if da != d else [])

    @functools.partial(
        pl.kernel,
        out_type=jax.ShapeDtypeStruct((NC * N, da), jnp.float32),
        mesh=mesh,
        scratch_types=[
            pltpu.VMEM((CPW // 2, CH), jnp.int32),
            pltpu.VMEM((CPW // 2, CH), jnp.int32),
            pltpu.VMEM((CPW // 2, CH), jnp.float32),
            pltpu.VMEM((CH, d), jnp.float32),
            pltpu.VMEM((CH, d), jnp.float32),
        ] + scaled_bufs + [
            pltpu.VMEM_SHARED((N, da), jnp.float32),
            pltpu.SemaphoreType.DMA,
            pltpu.SemaphoreType.DMA,
        ],
        compiler_params=pltpu.CompilerParams(needs_layout_passes=False),
    )
    def k(hp_hbm, row_hbm, col_hbm, ew_hbm, z_hbm, out_hbm,
          rowvs, colvs, ewvs, rows0, rows1, *rest):
        if da != d:
            sc0, sc1, acc, sem_g, sem_s = rest
        else:
            acc, sem_g, sem_s = rest
            sc0, sc1 = rows0, rows1
        cid = lax.axis_index("c")
        sid = lax.axis_index("s")
        w = (cid * NS + sid).astype(jnp.int32)
        sbase = pl.multiple_of(w * CPW, 8)
        rbase = pl.multiple_of(sid * RS_A, 8)

        @pl.when(sid < NS - 1)
        def _():
            pltpu.sync_copy(z_hbm.at[pl.ds(rbase, RS_A)],
                            acc.at[pl.ds(rbase, RS_A)])

        @pl.when(sid == NS - 1)
        def _():
            pltpu.sync_copy(z_hbm.at[pl.ds(rbase, RS_B)],
                            acc.at[pl.ds(rbase, RS_B)])

        plsc.subcore_barrier()
        HALF = CPW // 2

        def gather_desc(t, dst):
            return pltpu.make_async_copy(hp_hbm.at[rowvs.at[t]], dst, sem_g)

        def scatter_desc(t, src):
            return pltpu.make_async_copy(src, acc.at[colvs.at[t]], sem_s)

        def process(t, rb, sb, other_sb):
            gather_desc(t, rb).wait()

            @pl.when(t >= 1)
            def _():
                scatter_desc(t - jnp.int32(1), other_sb).wait()

            @pl.when(t < HALF - 1)
            def _():
                gather_desc(t + jnp.int32(1),
                            rows1 if rb is rows0 else rows0).start()

            tsplat = jnp.full((16,), 0, jnp.int32) + t

            def scale(_, j):
                s = plsc.load_gather(
                    ewvs, [tsplat, jnp.full((16,), 0, jnp.int32) + j])
                for t16 in range(da // 16):
                    sl = pl.ds(t16 * 16, 16)
                    sb[j, sl] = rb[j, sl] * s
                return j + jnp.int32(1)

            lax.fori_loop(0, CH, scale, jnp.int32(0), unroll=8)

            scatter_desc(t, sb).start(add=True)

        def run_half(h, carry):
            hbase = pl.multiple_of(sbase + h * jnp.int32(HALF), 8)
            pltpu.sync_copy(row_hbm.at[pl.ds(hbase, HALF)], rowvs)
            pltpu.sync_copy(col_hbm.at[pl.ds(hbase, HALF)], colvs)
            pltpu.sync_copy(ew_hbm.at[pl.ds(hbase, HALF)], ewvs)
            gather_desc(jnp.int32(0), rows0).start()

            def body(tt, c2):
                t = tt * jnp.int32(2)
                process(t, rows0, sc0, sc1)
                process(t + jnp.int32(1), rows1, sc1, sc0)
                return c2

            lax.fori_loop(jnp.int32(0), jnp.int32(HALF // 2), body,
                          jnp.int32(0))
            scatter_desc(jnp.int32(HALF - 1), sc1).wait()
            return carry

        lax.fori_loop(jnp.int32(0), jnp.int32(2), run_half, jnp.int32(0))
        plsc.subcore_barrier()

        obase = pl.multiple_of(cid * N + sid * RS_A, 8)

        @pl.when(sid < NS - 1)
        def _():
            pltpu.sync_copy(acc.at[pl.ds(rbase, RS_A)],
                            out_hbm.at[pl.ds(obase, RS_A)])

        @pl.when(sid == NS - 1)
        def _():
            pltpu.sync_copy(acc.at[pl.ds(rbase, RS_B)],
                            out_hbm.at[pl.ds(obase, RS_B)])

    return k(hp, rowc, colc, ewc, zeros_nd).reshape(NC, N, da)


BN = 400


def _l1_body(x_ref, w_ref, dp_ref, hp_ref, dv_ref):
    deg = dp_ref[0] + dp_ref[1] + jnp.float32(1.0)
    dinv = jnp.where(deg > 0, lax.rsqrt(deg), jnp.float32(0.0))
    h = jnp.dot(x_ref[...], w_ref[...], preferred_element_type=jnp.float32)
    hp_ref[...] = h * dinv
    dv_ref[...] = dinv


def _tc_layer1(x, W1, degp3):
    return pl.pallas_call(
        _l1_body,
        grid=(N // BN,),
        in_specs=[
            pl.BlockSpec((BN, 128), lambda i: (i, i * 0)),
            pl.BlockSpec((128, 128), lambda i: (i * 0, i * 0)),
            pl.BlockSpec((2, BN, 1), lambda i: (i * 0, i, i * 0)),
        ],
        out_specs=[
            pl.BlockSpec((BN, 128), lambda i: (i, i * 0)),
            pl.BlockSpec((BN, 1), lambda i: (i, i * 0)),
        ],
        out_shape=[
            jax.ShapeDtypeStruct((N, 128), jnp.float32),
            jax.ShapeDtypeStruct((N, 1), jnp.float32),
        ],
    )(x, W1, degp3)


def _l2_body(a_ref, hp_ref, d_ref, b_ref, w_ref, o_ref):
    agg = a_ref[0] + a_ref[1] + hp_ref[...]
    z = jax.nn.relu(agg * d_ref[...] + b_ref[...])
    h2 = jnp.dot(z, w_ref[...], preferred_element_type=jnp.float32)
    o_ref[...] = h2 * d_ref[...]


def _tc_layer2(accp1, hp1, dinv_col, b1_row, W2):
    return pl.pallas_call(
        _l2_body,
        grid=(N // BN,),
        in_specs=[
            pl.BlockSpec((2, BN, 128), lambda i: (i * 0, i, i * 0)),
            pl.BlockSpec((BN, 128), lambda i: (i, i * 0)),
            pl.BlockSpec((BN, 1), lambda i: (i, i * 0)),
            pl.BlockSpec((1, 128), lambda i: (i * 0, i * 0)),
            pl.BlockSpec((128, 128), lambda i: (i * 0, i * 0)),
        ],
        out_specs=pl.BlockSpec((BN, 128), lambda i: (i, i * 0)),
        out_shape=jax.ShapeDtypeStruct((N, 128), jnp.float32),
    )(accp1, hp1, dinv_col, b1_row, W2)


def _fin_body(a_ref, hp_ref, d_ref, b_ref, o_ref):
    agg = a_ref[0] + a_ref[1] + hp_ref[...]
    o_ref[...] = agg[:, :64] * d_ref[...] + b_ref[...]


def _tc_final(accp2, hp2, dinv_col, b2_row):
    return pl.pallas_call(
        _fin_body,
        grid=(N // BN,),
        in_specs=[
            pl.BlockSpec((2, BN, 128), lambda i: (i * 0, i, i * 0)),
            pl.BlockSpec((BN, 128), lambda i: (i, i * 0)),
            pl.BlockSpec((BN, 1), lambda i: (i, i * 0)),
            pl.BlockSpec((1, 64), lambda i: (i * 0, i * 0)),
        ],
        out_specs=pl.BlockSpec((BN, 64), lambda i: (i, i * 0)),
        out_shape=jax.ShapeDtypeStruct((N, 64), jnp.float32),
    )(accp2, hp2, dinv_col, b2_row)


def kernel(x, edge_index, edge_weight, W1, b1, W2, b2):
    row = edge_index[0].astype(jnp.int32)
    col = edge_index[1].astype(jnp.int32)
    ew = edge_weight.astype(jnp.float32)
    x = x.astype(jnp.float32)

    rowc, colc, ewc = _pad_edges(row, col, ew)

    zeros_n = jnp.zeros((N128,), jnp.float32)
    zeros_n128 = jnp.zeros((N, 128), jnp.float32)
    zeros_n64 = jnp.zeros((N, 64), jnp.float32)

    W2p = jnp.concatenate(
        [W2.astype(jnp.float32), jnp.zeros((128, 64), jnp.float32)], axis=1)

    degp = _sc_deg(colc, ewc, zeros_n)
    degp3 = degp.reshape(NC, N, 1)

    hp1, dinv_col = _tc_layer1(x, W1.astype(jnp.float32), degp3)
    accp1 = _sc_agg(hp1, rowc, colc, ewc, zeros_n128, 128)
    hp2 = _tc_layer2(accp1, hp1, dinv_col,
                     b1.astype(jnp.float32).reshape(1, 128),
                     W2p)
    accp2 = _sc_agg(hp2, rowc, colc, ewc, zeros_n128, 128)
    out = _tc_final(accp2, hp2, dinv_col,
                    b2.astype(jnp.float32).reshape(1, 64))
    return out


# speedup vs baseline: 1.0062x; 1.0046x over previous
import functools

import jax
import jax.numpy as jnp
from jax import lax
from jax.experimental import pallas as pl
from jax.experimental.pallas import tpu as pltpu
from jax.experimental.pallas import tpu_sc as plsc

N = 10000
E = 320000
NC = 2
NS = 16
NW = NC * NS
CH = 128
CPW = 80
E_PAD = NW * CPW * CH
RS_A = 632
RS_B = N - 15 * RS_A
N128 = 10112


def _pad_edges(row, col, ew):
    npad = E_PAD - E
    fill = (jnp.arange(npad, dtype=jnp.int32) * 37) % N
    rowp = jnp.concatenate([row, fill]).reshape(NW * CPW, CH)
    colp = jnp.concatenate([col, fill]).reshape(NW * CPW, CH)
    ewp = jnp.concatenate(
        [ew, jnp.zeros((npad,), jnp.float32)]).reshape(NW * CPW, CH)
    return rowp, colp, ewp


def _sc_deg(colc, ewc, zeros_n):
    mesh = plsc.VectorSubcoreMesh(core_axis_name="c", subcore_axis_name="s")

    @functools.partial(
        pl.kernel,
        out_type=jax.ShapeDtypeStruct((NC * N128,), jnp.float32),
        mesh=mesh,
        scratch_types=[
            pltpu.VMEM((CPW, CH), jnp.int32),
            pltpu.VMEM((CPW, CH), jnp.float32),
            pltpu.VMEM_SHARED((N128,), jnp.float32),
            pltpu.SemaphoreType.DMA,
        ],
    )
    def k(col_hbm, ew_hbm, z_hbm, out_hbm, colvs, ewvs, deg_sh, sem):
        cid = lax.axis_index("c")
        sid = lax.axis_index("s")
        w = (cid * NS + sid).astype(jnp.int32)
        sbase = pl.multiple_of(w * CPW, 8)

        @pl.when(sid == 0)
        def _():
            pltpu.sync_copy(z_hbm, deg_sh)

        pltpu.sync_copy(col_hbm.at[pl.ds(sbase, CPW)], colvs)
        pltpu.sync_copy(ew_hbm.at[pl.ds(sbase, CPW)], ewvs)
        plsc.subcore_barrier()

        def body(tt, carry):
            t0 = tt * jnp.int32(8)
            descs = [pltpu.make_async_copy(ewvs.at[t0 + j],
                                           deg_sh.at[colvs.at[t0 + j]], sem)
                     for j in range(8)]
            for dsc in descs:
                dsc.start(add=True)
            for dsc in descs:
                dsc.wait()
            return carry

        lax.fori_loop(jnp.int32(0), jnp.int32(CPW // 8), body, jnp.int32(0))
        plsc.subcore_barrier()

        @pl.when((sid == 0) & (cid == 0))
        def _():
            pltpu.sync_copy(deg_sh, out_hbm.at[pl.ds(0, N128)])

        @pl.when((sid == 0) & (cid == 1))
        def _():
            pltpu.sync_copy(deg_sh, out_hbm.at[pl.ds(N128, N128)])

    return k(colc, ewc, zeros_n).reshape(NC, N128)[:, :N]


def _sc_agg(hp, rowc, colc, ewc, zeros_nd, da):
    d = 128
    mesh = plsc.VectorSubcoreMesh(core_axis_name="c", subcore_axis_name="s")

    scaled_bufs = (
        [pltpu.VMEM((CH, da), jnp.float32),
         pltpu.VMEM((CH, da), jnp.float32)]
        if da != d else [])

    @functools.partial(
        pl.kernel,
        out_type=jax.ShapeDtypeStruct((NC * N, da), jnp.float32),
        mesh=mesh,
        scratch_types=[
            pltpu.VMEM((CPW // 2, CH), jnp.int32),
            pltpu.VMEM((CPW // 2, CH), jnp.int32),
            pltpu.VMEM((CPW // 2, CH), jnp.float32),
            pltpu.VMEM((CH, d), jnp.float32),
            pltpu.VMEM((CH, d), jnp.float32),
        ] + scaled_bufs + [
            pltpu.VMEM_SHARED((N, da), jnp.float32),
            pltpu.SemaphoreType.DMA,
            pltpu.SemaphoreType.DMA,
        ],
        compiler_params=pltpu.CompilerParams(needs_layout_passes=False),
    )
    def k(hp_hbm, row_hbm, col_hbm, ew_hbm, z_hbm, out_hbm,
          rowvs, colvs, ewvs, rows0, rows1, *rest):
        if da != d:
            sc0, sc1, acc, sem_g, sem_s = rest
        else:
            acc, sem_g, sem_s = rest
            sc0, sc1 = rows0, rows1
        cid = lax.axis_index("c")
        sid = lax.axis_index("s")
        w = (cid * NS + sid).astype(jnp.int32)
        sbase = pl.multiple_of(w * CPW, 8)
        rbase = pl.multiple_of(sid * RS_A, 8)

        @pl.when(sid < NS - 1)
        def _():
            pltpu.sync_copy(z_hbm.at[pl.ds(rbase, RS_A)],
                            acc.at[pl.ds(rbase, RS_A)])

        @pl.when(sid == NS - 1)
        def _():
            pltpu.sync_copy(z_hbm.at[pl.ds(rbase, RS_B)],
                            acc.at[pl.ds(rbase, RS_B)])

        plsc.subcore_barrier()
        HALF = CPW // 2

        def gather_desc(t, dst):
            return pltpu.make_async_copy(hp_hbm.at[rowvs.at[t]], dst, sem_g)

        def scatter_desc(t, src):
            return pltpu.make_async_copy(src, acc.at[colvs.at[t]], sem_s)

        def process(t, rb, sb, other_sb):
            gather_desc(t, rb).wait()

            @pl.when(t >= 1)
            def _():
                scatter_desc(t - jnp.int32(1), other_sb).wait()

            @pl.when(t < HALF - 1)
            def _():
                gather_desc(t + jnp.int32(1),
                            rows1 if rb is rows0 else rows0).start()

            tsplat = jnp.full((16,), 0, jnp.int32) + t

            def scale(_, j):
                s = plsc.load_gather(
                    ewvs, [tsplat, jnp.full((16,), 0, jnp.int32) + j])
                for t16 in range(da // 16):
                    sl = pl.ds(t16 * 16, 16)
                    sb[j, sl] = rb[j, sl] * s
                return j + jnp.int32(1)

            lax.fori_loop(0, CH, scale, jnp.int32(0), unroll=8)

            scatter_desc(t, sb).start(add=True)

        def run_half(h, carry):
            hbase = pl.multiple_of(sbase + h * jnp.int32(HALF), 8)
            pltpu.sync_copy(row_hbm.at[pl.ds(hbase, HALF)], rowvs)
            pltpu.sync_copy(col_hbm.at[pl.ds(hbase, HALF)], colvs)
            pltpu.sync_copy(ew_hbm.at[pl.ds(hbase, HALF)], ewvs)
            gather_desc(jnp.int32(0), rows0).start()

            def body(tt, c2):
                t = tt * jnp.int32(2)
                process(t, rows0, sc0, sc1)
                process(t + jnp.int32(1), rows1, sc1, sc0)
                return c2

            lax.fori_loop(jnp.int32(0), jnp.int32(HALF // 2), body,
                          jnp.int32(0))
            scatter_desc(jnp.int32(HALF - 1), sc1).wait()
            return carry

        lax.fori_loop(jnp.int32(0), jnp.int32(2), run_half, jnp.int32(0))
        plsc.subcore_barrier()

        obase = pl.multiple_of(cid * N + sid * RS_A, 8)

        @pl.when(sid < NS - 1)
        def _():
            pltpu.sync_copy(acc.at[pl.ds(rbase, RS_A)],
                            out_hbm.at[pl.ds(obase, RS_A)])

        @pl.when(sid == NS - 1)
        def _():
            pltpu.sync_copy(acc.at[pl.ds(rbase, RS_B)],
                            out_hbm.at[pl.ds(obase, RS_B)])

    return k(hp, rowc, colc, ewc, zeros_nd).reshape(NC, N, da)


BN = 400


def _l1_body(x_ref, w_ref, dp_ref, hp_ref, dv_ref):
    deg = dp_ref[0] + dp_ref[1] + jnp.float32(1.0)
    dinv = jnp.where(deg > 0, lax.rsqrt(deg), jnp.float32(0.0))
    h = jnp.dot(x_ref[...], w_ref[...], preferred_element_type=jnp.float32)
    hp_ref[...] = h * dinv
    dv_ref[...] = dinv


def _tc_layer1(x, W1, degp3):
    return pl.pallas_call(
        _l1_body,
        grid=(N // BN,),
        in_specs=[
            pl.BlockSpec((BN, 128), lambda i: (i, i * 0)),
            pl.BlockSpec((128, 128), lambda i: (i * 0, i * 0)),
            pl.BlockSpec((2, BN, 1), lambda i: (i * 0, i, i * 0)),
        ],
        out_specs=[
            pl.BlockSpec((BN, 128), lambda i: (i, i * 0)),
            pl.BlockSpec((BN, 1), lambda i: (i, i * 0)),
        ],
        out_shape=[
            jax.ShapeDtypeStruct((N, 128), jnp.float32),
            jax.ShapeDtypeStruct((N, 1), jnp.float32),
        ],
    )(x, W1, degp3)


def _l2_body(a_ref, hp_ref, d_ref, b_ref, w_ref, o_ref):
    agg = a_ref[0] + a_ref[1] + hp_ref[...]
    z = jax.nn.relu(agg * d_ref[...] + b_ref[...])
    h2 = jnp.dot(z, w_ref[...], preferred_element_type=jnp.float32)
    o_ref[...] = h2 * d_ref[...]


def _tc_layer2(accp1, hp1, dinv_col, b1_row, W2):
    return pl.pallas_call(
        _l2_body,
        grid=(N // BN,),
        in_specs=[
            pl.BlockSpec((2, BN, 128), lambda i: (i * 0, i, i * 0)),
            pl.BlockSpec((BN, 128), lambda i: (i, i * 0)),
            pl.BlockSpec((BN, 1), lambda i: (i, i * 0)),
            pl.BlockSpec((1, 128), lambda i: (i * 0, i * 0)),
            pl.BlockSpec((128, 128), lambda i: (i * 0, i * 0)),
        ],
        out_specs=pl.BlockSpec((BN, 128), lambda i: (i, i * 0)),
        out_shape=jax.ShapeDtypeStruct((N, 128), jnp.float32),
    )(accp1, hp1, dinv_col, b1_row, W2)


def _fin_body(a_ref, hp_ref, d_ref, b_ref, o_ref):
    agg = a_ref[0] + a_ref[1] + hp_ref[...]
    o_ref[...] = agg[:, :64] * d_ref[...] + b_ref[...]


def _tc_final(accp2, hp2, dinv_col, b2_row):
    return pl.pallas_call(
        _fin_body,
        grid=(N // BN,),
        in_specs=[
            pl.BlockSpec((2, BN, 128), lambda i: (i * 0, i, i * 0)),
            pl.BlockSpec((BN, 128), lambda i: (i, i * 0)),
            pl.BlockSpec((BN, 1), lambda i: (i, i * 0)),
            pl.BlockSpec((1, 64), lambda i: (i * 0, i * 0)),
        ],
        out_specs=pl.BlockSpec((BN, 64), lambda i: (i, i * 0)),
        out_shape=jax.ShapeDtypeStruct((N, 64), jnp.float32),
    )(accp2, hp2, dinv_col, b2_row)


def kernel(x, edge_index, edge_weight, W1, b1, W2, b2):
    row = edge_index[0].astype(jnp.int32)
    col = edge_index[1].astype(jnp.int32)
    ew = edge_weight.astype(jnp.float32)
    x = x.astype(jnp.float32)

    rowc, colc, ewc = _pad_edges(row, col, ew)

    zeros_n = jnp.zeros((N128,), jnp.float32)
    zeros_n128 = jnp.zeros((N, 128), jnp.float32)
    zeros_n64 = jnp.zeros((N, 64), jnp.float32)

    W2p = jnp.concatenate(
        [W2.astype(jnp.float32), jnp.zeros((128, 64), jnp.float32)], axis=1)

    degp = _sc_deg(colc, ewc, zeros_n)
    degp3 = degp.reshape(NC, N, 1)

    hp1, dinv_col = _tc_layer1(x, W1.astype(jnp.float32), degp3)
    accp1 = _sc_agg(hp1, rowc, colc, ewc, zeros_n128, 128)
    hp2 = _tc_layer2(accp1, hp1, dinv_col,
                     b1.astype(jnp.float32).reshape(1, 128),
                     W2p)
    accp2 = _sc_agg(hp2, rowc, colc, ewc, zeros_n128, 128)
    out = _tc_final(accp2, hp2, dinv_col,
                    b2.astype(jnp.float32).reshape(1, 64))
    return out
